# baseline (speedup 1.0000x reference)
import functools

import jax
import jax.numpy as jnp
from jax import lax
from jax.experimental import pallas as pl
from jax.experimental.pallas import tpu as pltpu
from jax.experimental.pallas import tpu_sc as plsc

B = 16384
E = 16
NC = 2
NS = 16
NW = NC * NS
BPW = B // NW
CH = 128
NCH = BPW // CH
V = 1000000
VA = 999936
NT = V - VA
PAD = 1000448
CHW = 32256
NLCH = VA // CHW


def _sc_linearize_bias(u_tabT, r_tabT, u_idx2d, r_idx2d, u_bias, r_bias):
    mesh = plsc.VectorSubcoreMesh(core_axis_name="c", subcore_axis_name="s")

    @functools.partial(
        pl.kernel,
        mesh=mesh,
        out_type=(
            jax.ShapeDtypeStruct((E * PAD,), jnp.float32),
            jax.ShapeDtypeStruct((E * PAD,), jnp.float32),
            jax.ShapeDtypeStruct((B,), jnp.float32),
        ),
        scratch_types=[
            pltpu.VMEM((2, CHW), jnp.float32),
            pltpu.VMEM((NCH, CH), jnp.int32),
            pltpu.VMEM((NCH, CH), jnp.int32),
            pltpu.VMEM((BPW,), jnp.float32),
            pltpu.VMEM((BPW,), jnp.float32),
            pltpu.VMEM((BPW,), jnp.float32),
            pltpu.SemaphoreType.DMA,
            pltpu.SemaphoreType.DMA,
            pltpu.SemaphoreType.DMA,
            pltpu.SemaphoreType.DMA,
        ],
        compiler_params=pltpu.CompilerParams(needs_layout_passes=False),
    )
    def k(u_tab_hbm, r_tab_hbm, u_idx_hbm, r_idx_hbm, u_bias_hbm, r_bias_hbm,
          u_out_hbm, r_out_hbm, ubrb_hbm,
          buf, idx_u, idx_r, ub_v, rb_v, ubrb_v, sem_in, s_w0, s_w1, sem_b):
        wid = lax.axis_index("s") * NC + lax.axis_index("c")
        lane = lax.rem(wid, E)
        base = pl.multiple_of(wid * BPW, 8)
        row0 = wid * NCH
        wsems = (s_w0, s_w1)

        pltpu.sync_copy(u_idx_hbm.at[pl.ds(row0, NCH)], idx_u)
        pltpu.sync_copy(r_idx_hbm.at[pl.ds(row0, NCH)], idx_r)
        bias_copies = []
        for j in range(NCH):
            sl = pl.ds(j * CH, CH)
            bias_copies.append(
                pltpu.async_copy(u_bias_hbm.at[idx_u.at[j]], ub_v.at[sl], sem_b))
            bias_copies.append(
                pltpu.async_copy(r_bias_hbm.at[idx_r.at[j]], rb_v.at[sl], sem_b))

        def do_table(tab, out):
            reads = [None, None]
            writes = [None, None]
            reads[0] = pltpu.async_copy(
                tab.at[lane, pl.ds(0, CHW)], buf.at[0], sem_in)
            for c in range(NLCH):
                b = c % 2
                reads[b].wait()
                if c + 1 < NLCH:
                    nb = (c + 1) % 2
                    if writes[nb] is not None:
                        writes[nb].wait()
                    reads[nb] = pltpu.async_copy(
                        tab.at[lane, pl.ds((c + 1) * CHW, CHW)], buf.at[nb], sem_in)
                writes[b] = pltpu.async_copy(
                    buf.at[b], out.at[pl.ds(lane * PAD + c * CHW, CHW)], wsems[b])
            for w in writes:
                if w is not None:
                    w.wait()

        @pl.when(wid < E)
        def _():
            do_table(u_tab_hbm, u_out_hbm)

        @pl.when(wid >= E)
        def _():
            do_table(r_tab_hbm, r_out_hbm)

        for c in bias_copies:
            c.wait()
        for i in range(BPW // E):
            sl = pl.ds(i * E, E)
            ubrb_v[sl] = ub_v[sl] + rb_v[sl]
        pltpu.sync_copy(ubrb_v, ubrb_hbm.at[pl.ds(base, BPW)])

    return k(u_tabT, r_tabT, u_idx2d, r_idx2d, u_bias, r_bias)


def _sc_gather_dot(u_lidx, r_lidx, u_idx2d, r_idx2d, u_lin, r_lin,
                   u_tail, r_tail):
    mesh = plsc.VectorSubcoreMesh(core_axis_name="c", subcore_axis_name="s")

    @functools.partial(
        pl.kernel,
        mesh=mesh,
        out_type=jax.ShapeDtypeStruct((NW * E,), jnp.float32),
        scratch_types=[
            pltpu.VMEM((E, BPW), jnp.int32),
            pltpu.VMEM((E, BPW), jnp.int32),
            pltpu.VMEM((NCH, CH), jnp.int32),
            pltpu.VMEM((NCH, CH), jnp.int32),
            pltpu.VMEM((E, BPW), jnp.float32),
            pltpu.VMEM((E, BPW), jnp.float32),
            pltpu.VMEM((NT, E), jnp.float32),
            pltpu.VMEM((NT, E), jnp.float32),
            pltpu.VMEM((E,), jnp.float32),
            pltpu.SemaphoreType.DMA,
            pltpu.SemaphoreType.DMA,
            pltpu.SemaphoreType.DMA,
            pltpu.SemaphoreType.DMA,
        ],
        compiler_params=pltpu.CompilerParams(
            use_tc_tiling_on_sc=False, needs_layout_passes=False),
    )
    def k(u_lidx_hbm, r_lidx_hbm, u_idx_hbm, r_idx_hbm, u_lin_hbm, r_lin_hbm,
          u_tail_hbm, r_tail_hbm, partial_hbm,
          lidx_u, lidx_r, idx_u, idx_r, u_vals, r_vals, ut_v, rt_v,
          acc_v, sem0, sem1, sem2, sem3):
        wid = lax.axis_index("s") * NC + lax.axis_index("c")
        row0 = wid * NCH
        sems = (sem0, sem1, sem2, sem3)

        pltpu.sync_copy(u_lidx_hbm.at[wid], lidx_u)
        pltpu.sync_copy(r_lidx_hbm.at[wid], lidx_r)
        pltpu.sync_copy(u_idx_hbm.at[pl.ds(row0, NCH)], idx_u)
        pltpu.sync_copy(r_idx_hbm.at[pl.ds(row0, NCH)], idx_r)
        pltpu.sync_copy(u_tail_hbm, ut_v)
        pltpu.sync_copy(r_tail_hbm, rt_v)

        waves = []
        for j in range(NCH):
            sl = pl.ds(j * CH, CH)
            wave = []
            for l in range(E):
                wave.append(pltpu.async_copy(
                    u_lin_hbm.at[lidx_u.at[l, sl]], u_vals.at[l, sl], sems[j]))
                wave.append(pltpu.async_copy(
                    r_lin_hbm.at[lidx_r.at[l, sl]], r_vals.at[l, sl], sems[j]))
            waves.append(wave)

        acc = jnp.zeros((E,), jnp.float32)
        for j in range(NCH):
            for c in waves[j]:
                c.wait()
            for i8 in range(CH // E):
                g = pl.ds(j * CH + i8 * E, E)
                gi = pl.ds(i8 * E, E)
                iu = idx_u[j, gi]
                ir = idx_r[j, gi]
                mu = iu >= VA
                mr = ir >= VA
                tu = jnp.maximum(iu - VA, 0)
                tr = jnp.maximum(ir - VA, 0)
                for l in range(E):
                    lcol = jnp.full((E,), l, jnp.int32)
                    uv = jnp.where(mu, plsc.load_gather(ut_v, [tu, lcol]),
                                   u_vals[l, g])
                    rv = jnp.where(mr, plsc.load_gather(rt_v, [tr, lcol]),
                                   r_vals[l, g])
                    acc = acc + uv * rv
        acc_v[...] = acc
        pltpu.sync_copy(acc_v, partial_hbm.at[pl.ds(pl.multiple_of(wid * E, 8), E)])

    return k(u_lidx, r_lidx, u_idx2d, r_idx2d, u_lin, r_lin, u_tail, r_tail)


def _tc_finish(partials_2d, ubrb_2d):
    def body(p_ref, x_ref, o_ref):
        s = jnp.sum(p_ref[...])
        o_ref[...] = jax.nn.sigmoid(x_ref[...] + s)

    return pl.pallas_call(
        body,
        out_shape=jax.ShapeDtypeStruct(ubrb_2d.shape, jnp.float32),
    )(partials_2d, ubrb_2d)


def kernel(inputs, user_embedding, user_bias, resto_embedding, resto_bias):
    idx = inputs.astype(jnp.int32)
    u_idx = idx[:, 0]
    r_idx = idx[:, 1]
    lane_off = (jnp.arange(E, dtype=jnp.int32) * PAD).reshape(1, E, 1)
    u_lidx = jnp.minimum(u_idx, VA - 1).reshape(NW, 1, BPW) + lane_off
    r_lidx = jnp.minimum(r_idx, VA - 1).reshape(NW, 1, BPW) + lane_off
    u_idx2d = u_idx.reshape(B // CH, CH)
    r_idx2d = r_idx.reshape(B // CH, CH)
    u_lin, r_lin, ubrb = _sc_linearize_bias(
        user_embedding.T, resto_embedding.T, u_idx2d, r_idx2d,
        user_bias.reshape(-1), resto_bias.reshape(-1))
    partials = _sc_gather_dot(
        u_lidx, r_lidx, u_idx2d, r_idx2d, u_lin, r_lin,
        user_embedding[VA:, :], resto_embedding[VA:, :])
    out = _tc_finish(partials.reshape(NW * E // 128, 128), ubrb.reshape(B // 128, 128))
    return out.reshape(B, 1)


# speedup vs baseline: 1.4855x; 1.4855x over previous
import functools

import jax
import jax.numpy as jnp
from jax import lax
from jax.experimental import pallas as pl
from jax.experimental.pallas import tpu as pltpu
from jax.experimental.pallas import tpu_sc as plsc

B = 16384
E = 16
NC = 2
NS = 16
NW = NC * NS
BPW = B // NW
CH = 128
NCH = BPW // CH
V = 1000000
VA = 999936
NT = V - VA
PAD = 1000448
CHW = 32256
NLCH = VA // CHW


def _sc_linearize(u_tabT, r_tabT):
    mesh = plsc.VectorSubcoreMesh(core_axis_name="c", subcore_axis_name="s")

    @functools.partial(
        pl.kernel,
        mesh=mesh,
        out_type=(
            jax.ShapeDtypeStruct((E * PAD,), jnp.float32),
            jax.ShapeDtypeStruct((E * PAD,), jnp.float32),
        ),
        scratch_types=[
            pltpu.VMEM((2, CHW), jnp.float32),
            pltpu.SemaphoreType.DMA,
            pltpu.SemaphoreType.DMA,
            pltpu.SemaphoreType.DMA,
        ],
        compiler_params=pltpu.CompilerParams(needs_layout_passes=False),
    )
    def k(u_tab_hbm, r_tab_hbm, u_out_hbm, r_out_hbm, buf, sem_in, s_w0, s_w1):
        wid = lax.axis_index("s") * NC + lax.axis_index("c")
        lane = lax.rem(wid, E)
        wsems = (s_w0, s_w1)

        def do_table(tab, out):
            reads = [None, None]
            writes = [None, None]
            reads[0] = pltpu.async_copy(
                tab.at[lane, pl.ds(0, CHW)], buf.at[0], sem_in)
            for c in range(NLCH):
                b = c % 2
                reads[b].wait()
                if c + 1 < NLCH:
                    nb = (c + 1) % 2
                    if writes[nb] is not None:
                        writes[nb].wait()
                    reads[nb] = pltpu.async_copy(
                        tab.at[lane, pl.ds((c + 1) * CHW, CHW)], buf.at[nb], sem_in)
                writes[b] = pltpu.async_copy(
                    buf.at[b], out.at[pl.ds(lane * PAD + c * CHW, CHW)], wsems[b])
            for w in writes:
                if w is not None:
                    w.wait()

        @pl.when(wid < E)
        def _():
            do_table(u_tab_hbm, u_out_hbm)

        @pl.when(wid >= E)
        def _():
            do_table(r_tab_hbm, r_out_hbm)

    return k(u_tabT, r_tabT)


def _sc_gather_dot(u_lidx, r_lidx, u_idx2d, r_idx2d, u_lin, r_lin,
                   u_tail, r_tail, u_bias, r_bias):
    mesh = plsc.VectorSubcoreMesh(core_axis_name="c", subcore_axis_name="s")

    @functools.partial(
        pl.kernel,
        mesh=mesh,
        out_type=(
            jax.ShapeDtypeStruct((NW * E,), jnp.float32),
            jax.ShapeDtypeStruct((B,), jnp.float32),
        ),
        scratch_types=[
            pltpu.VMEM((E, BPW), jnp.int32),
            pltpu.VMEM((E, BPW), jnp.int32),
            pltpu.VMEM((NCH, CH), jnp.int32),
            pltpu.VMEM((NCH, CH), jnp.int32),
            pltpu.VMEM((E, BPW), jnp.float32),
            pltpu.VMEM((E, BPW), jnp.float32),
            pltpu.VMEM((NT, E), jnp.float32),
            pltpu.VMEM((NT, E), jnp.float32),
            pltpu.VMEM((BPW,), jnp.float32),
            pltpu.VMEM((BPW,), jnp.float32),
            pltpu.VMEM((BPW,), jnp.float32),
            pltpu.VMEM((E,), jnp.float32),
            pltpu.SemaphoreType.DMA,
        ],
        compiler_params=pltpu.CompilerParams(
            use_tc_tiling_on_sc=False, needs_layout_passes=False),
    )
    def k(u_lidx_hbm, r_lidx_hbm, u_idx_hbm, r_idx_hbm, u_lin_hbm, r_lin_hbm,
          u_tail_hbm, r_tail_hbm, u_bias_hbm, r_bias_hbm, partial_hbm, ubrb_hbm,
          lidx_u, lidx_r, idx_u, idx_r, u_vals, r_vals, ut_v, rt_v, ub_v, rb_v,
          ubrb_v, acc_v, sem):
        wid = lax.axis_index("s") * NC + lax.axis_index("c")
        base = pl.multiple_of(wid * BPW, 8)
        row0 = wid * NCH

        pltpu.sync_copy(u_lidx_hbm.at[wid], lidx_u)
        pltpu.sync_copy(r_lidx_hbm.at[wid], lidx_r)
        pltpu.sync_copy(u_idx_hbm.at[pl.ds(row0, NCH)], idx_u)
        pltpu.sync_copy(r_idx_hbm.at[pl.ds(row0, NCH)], idx_r)
        pltpu.sync_copy(u_tail_hbm, ut_v)
        pltpu.sync_copy(r_tail_hbm, rt_v)

        for j in range(NCH):
            sl = pl.ds(j * CH, CH)
            copies = []
            for l in range(E):
                copies.append(pltpu.async_copy(
                    u_lin_hbm.at[lidx_u.at[l, sl]], u_vals.at[l, sl], sem))
                copies.append(pltpu.async_copy(
                    r_lin_hbm.at[lidx_r.at[l, sl]], r_vals.at[l, sl], sem))
            copies.append(pltpu.async_copy(u_bias_hbm.at[idx_u.at[j]], ub_v.at[sl], sem))
            copies.append(pltpu.async_copy(r_bias_hbm.at[idx_r.at[j]], rb_v.at[sl], sem))
            for c in copies:
                c.wait()

        acc = jnp.zeros((E,), jnp.float32)
        for j in range(NCH):
            for i8 in range(CH // E):
                g = pl.ds(j * CH + i8 * E, E)
                gi = pl.ds(i8 * E, E)
                iu = idx_u[j, gi]
                ir = idx_r[j, gi]
                mu = iu >= VA
                mr = ir >= VA
                tu = jnp.maximum(iu - VA, 0)
                tr = jnp.maximum(ir - VA, 0)
                for l in range(E):
                    lcol = jnp.full((E,), l, jnp.int32)
                    uv = jnp.where(mu, plsc.load_gather(ut_v, [tu, lcol]),
                                   u_vals[l, g])
                    rv = jnp.where(mr, plsc.load_gather(rt_v, [tr, lcol]),
                                   r_vals[l, g])
                    acc = acc + uv * rv
        acc_v[...] = acc
        pltpu.sync_copy(acc_v, partial_hbm.at[pl.ds(pl.multiple_of(wid * E, 8), E)])

        for i in range(BPW // E):
            sl = pl.ds(i * E, E)
            ubrb_v[sl] = ub_v[sl] + rb_v[sl]
        pltpu.sync_copy(ubrb_v, ubrb_hbm.at[pl.ds(base, BPW)])

    return k(u_lidx, r_lidx, u_idx2d, r_idx2d, u_lin, r_lin,
             u_tail, r_tail, u_bias, r_bias)


def _tc_finish(partials_2d, ubrb_2d):
    def body(p_ref, x_ref, o_ref):
        s = jnp.sum(p_ref[...])
        o_ref[...] = jax.nn.sigmoid(x_ref[...] + s)

    return pl.pallas_call(
        body,
        out_shape=jax.ShapeDtypeStruct(ubrb_2d.shape, jnp.float32),
    )(partials_2d, ubrb_2d)


def kernel(inputs, user_embedding, user_bias, resto_embedding, resto_bias):
    idx = inputs.astype(jnp.int32)
    u_idx = idx[:, 0]
    r_idx = idx[:, 1]
    lane_off = (jnp.arange(E, dtype=jnp.int32) * PAD).reshape(1, E, 1)
    u_lidx = jnp.minimum(u_idx, VA - 1).reshape(NW, 1, BPW) + lane_off
    r_lidx = jnp.minimum(r_idx, VA - 1).reshape(NW, 1, BPW) + lane_off
    u_lin, r_lin = _sc_linearize(user_embedding.T, resto_embedding.T)
    partials, ubrb = _sc_gather_dot(
        u_lidx, r_lidx,
        u_idx.reshape(B // CH, CH), r_idx.reshape(B // CH, CH),
        u_lin, r_lin,
        user_embedding[VA:, :], resto_embedding[VA:, :],
        user_bias.reshape(-1), resto_bias.reshape(-1))
    out = _tc_finish(partials.reshape(NW * E // 128, 128), ubrb.reshape(B // 128, 128))
    return out.reshape(B, 1)


# speedup vs baseline: 1.5390x; 1.0361x over previous
import functools

import jax
import jax.numpy as jnp
from jax import lax
from jax.experimental import pallas as pl
from jax.experimental.pallas import tpu as pltpu
from jax.experimental.pallas import tpu_sc as plsc

B = 16384
E = 16
NC = 2
NS = 16
NW = NC * NS
BPW = B // NW
CH = 128
NCH = BPW // CH
V = 1000000
VA = 999936
NT = V - VA
PAD = 1000448
CHW = 32256
NLCH = VA // CHW


def _sc_linearize(u_tabT, r_tabT):
    mesh = plsc.VectorSubcoreMesh(core_axis_name="c", subcore_axis_name="s")

    @functools.partial(
        pl.kernel,
        mesh=mesh,
        out_type=(
            jax.ShapeDtypeStruct((E * PAD,), jnp.float32),
            jax.ShapeDtypeStruct((E * PAD,), jnp.float32),
        ),
        scratch_types=[
            pltpu.VMEM((2, CHW), jnp.float32),
            pltpu.SemaphoreType.DMA,
            pltpu.SemaphoreType.DMA,
            pltpu.SemaphoreType.DMA,
        ],
        compiler_params=pltpu.CompilerParams(needs_layout_passes=False),
    )
    def k(u_tab_hbm, r_tab_hbm, u_out_hbm, r_out_hbm, buf, sem_in, s_w0, s_w1):
        wid = lax.axis_index("s") * NC + lax.axis_index("c")
        lane = lax.rem(wid, E)
        wsems = (s_w0, s_w1)

        def do_table(tab, out):
            reads = [None, None]
            writes = [None, None]
            reads[0] = pltpu.async_copy(
                tab.at[lane, pl.ds(0, CHW)], buf.at[0], sem_in)
            for c in range(NLCH):
                b = c % 2
                reads[b].wait()
                if c + 1 < NLCH:
                    nb = (c + 1) % 2
                    if writes[nb] is not None:
                        writes[nb].wait()
                    reads[nb] = pltpu.async_copy(
                        tab.at[lane, pl.ds((c + 1) * CHW, CHW)], buf.at[nb], sem_in)
                writes[b] = pltpu.async_copy(
                    buf.at[b], out.at[pl.ds(lane * PAD + c * CHW, CHW)], wsems[b])
            for w in writes:
                if w is not None:
                    w.wait()

        @pl.when(wid < E)
        def _():
            do_table(u_tab_hbm, u_out_hbm)

        @pl.when(wid >= E)
        def _():
            do_table(r_tab_hbm, r_out_hbm)

    return k(u_tabT, r_tabT)


def _sc_gather_dot(u_lidx, r_lidx, u_idx2d, r_idx2d, u_lin, r_lin,
                   u_tail, r_tail, u_bias, r_bias):
    mesh = plsc.VectorSubcoreMesh(core_axis_name="c", subcore_axis_name="s")

    @functools.partial(
        pl.kernel,
        mesh=mesh,
        out_type=(
            jax.ShapeDtypeStruct((NW * E,), jnp.float32),
            jax.ShapeDtypeStruct((B,), jnp.float32),
        ),
        scratch_types=[
            pltpu.VMEM((E, BPW), jnp.int32),
            pltpu.VMEM((E, BPW), jnp.int32),
            pltpu.VMEM((NCH, CH), jnp.int32),
            pltpu.VMEM((NCH, CH), jnp.int32),
            pltpu.VMEM((E, BPW), jnp.float32),
            pltpu.VMEM((E, BPW), jnp.float32),
            pltpu.VMEM((NT, E), jnp.float32),
            pltpu.VMEM((NT, E), jnp.float32),
            pltpu.VMEM((BPW,), jnp.float32),
            pltpu.VMEM((BPW,), jnp.float32),
            pltpu.VMEM((BPW,), jnp.float32),
            pltpu.VMEM((E,), jnp.float32),
            pltpu.SemaphoreType.DMA,
            pltpu.SemaphoreType.DMA,
            pltpu.SemaphoreType.DMA,
            pltpu.SemaphoreType.DMA,
            pltpu.SemaphoreType.DMA,
        ],
        compiler_params=pltpu.CompilerParams(
            use_tc_tiling_on_sc=False, needs_layout_passes=False),
    )
    def k(u_lidx_hbm, r_lidx_hbm, u_idx_hbm, r_idx_hbm, u_lin_hbm, r_lin_hbm,
          u_tail_hbm, r_tail_hbm, u_bias_hbm, r_bias_hbm, partial_hbm, ubrb_hbm,
          lidx_u, lidx_r, idx_u, idx_r, u_vals, r_vals, ut_v, rt_v, ub_v, rb_v,
          ubrb_v, acc_v, sem, sem_c0, sem_c1, sem_c2, sem_c3):
        wid = lax.axis_index("s") * NC + lax.axis_index("c")
        base = pl.multiple_of(wid * BPW, 8)
        row0 = wid * NCH

        pltpu.sync_copy(u_lidx_hbm.at[wid], lidx_u)
        pltpu.sync_copy(r_lidx_hbm.at[wid], lidx_r)
        pltpu.sync_copy(u_idx_hbm.at[pl.ds(row0, NCH)], idx_u)
        pltpu.sync_copy(r_idx_hbm.at[pl.ds(row0, NCH)], idx_r)
        pltpu.sync_copy(u_tail_hbm, ut_v)
        pltpu.sync_copy(r_tail_hbm, rt_v)

        csems = (sem_c0, sem_c1, sem_c2, sem_c3)
        bias_copies = []
        waves = []
        for j in range(NCH):
            sl = pl.ds(j * CH, CH)
            bias_copies.append(
                pltpu.async_copy(u_bias_hbm.at[idx_u.at[j]], ub_v.at[sl], sem))
            bias_copies.append(
                pltpu.async_copy(r_bias_hbm.at[idx_r.at[j]], rb_v.at[sl], sem))
            wave = []
            for l in range(E):
                wave.append(pltpu.async_copy(
                    u_lin_hbm.at[lidx_u.at[l, sl]], u_vals.at[l, sl], csems[j]))
                wave.append(pltpu.async_copy(
                    r_lin_hbm.at[lidx_r.at[l, sl]], r_vals.at[l, sl], csems[j]))
            waves.append(wave)

        acc = jnp.zeros((E,), jnp.float32)
        for j in range(NCH):
            for c in waves[j]:
                c.wait()
            for i8 in range(CH // E):
                g = pl.ds(j * CH + i8 * E, E)
                gi = pl.ds(i8 * E, E)
                iu = idx_u[j, gi]
                ir = idx_r[j, gi]
                mu = iu >= VA
                mr = ir >= VA
                tu = jnp.maximum(iu - VA, 0)
                tr = jnp.maximum(ir - VA, 0)
                for l in range(E):
                    lcol = jnp.full((E,), l, jnp.int32)
                    uv = jnp.where(mu, plsc.load_gather(ut_v, [tu, lcol]),
                                   u_vals[l, g])
                    rv = jnp.where(mr, plsc.load_gather(rt_v, [tr, lcol]),
                                   r_vals[l, g])
                    acc = acc + uv * rv
        acc_v[...] = acc
        pltpu.sync_copy(acc_v, partial_hbm.at[pl.ds(pl.multiple_of(wid * E, 8), E)])

        for c in bias_copies:
            c.wait()
        for i in range(BPW // E):
            sl = pl.ds(i * E, E)
            ubrb_v[sl] = ub_v[sl] + rb_v[sl]
        pltpu.sync_copy(ubrb_v, ubrb_hbm.at[pl.ds(base, BPW)])

    return k(u_lidx, r_lidx, u_idx2d, r_idx2d, u_lin, r_lin,
             u_tail, r_tail, u_bias, r_bias)


def _tc_finish(partials_2d, ubrb_2d):
    def body(p_ref, x_ref, o_ref):
        s = jnp.sum(p_ref[...])
        o_ref[...] = jax.nn.sigmoid(x_ref[...] + s)

    return pl.pallas_call(
        body,
        out_shape=jax.ShapeDtypeStruct(ubrb_2d.shape, jnp.float32),
    )(partials_2d, ubrb_2d)


def kernel(inputs, user_embedding, user_bias, resto_embedding, resto_bias):
    idx = inputs.astype(jnp.int32)
    u_idx = idx[:, 0]
    r_idx = idx[:, 1]
    lane_off = (jnp.arange(E, dtype=jnp.int32) * PAD).reshape(1, E, 1)
    u_lidx = jnp.minimum(u_idx, VA - 1).reshape(NW, 1, BPW) + lane_off
    r_lidx = jnp.minimum(r_idx, VA - 1).reshape(NW, 1, BPW) + lane_off
    u_lin, r_lin = _sc_linearize(user_embedding.T, resto_embedding.T)
    partials, ubrb = _sc_gather_dot(
        u_lidx, r_lidx,
        u_idx.reshape(B // CH, CH), r_idx.reshape(B // CH, CH),
        u_lin, r_lin,
        user_embedding[VA:, :], resto_embedding[VA:, :],
        user_bias.reshape(-1), resto_bias.reshape(-1))
    out = _tc_finish(partials.reshape(NW * E // 128, 128), ubrb.reshape(B // 128, 128))
    return out.reshape(B, 1)


# speedup vs baseline: 1.5398x; 1.0005x over previous
import functools

import jax
import jax.numpy as jnp
from jax import lax
from jax.experimental import pallas as pl
from jax.experimental.pallas import tpu as pltpu
from jax.experimental.pallas import tpu_sc as plsc

B = 16384
E = 16
NC = 2
NS = 16
NW = NC * NS
BPW = B // NW
CH = 128
NCH = BPW // CH
V = 1000000
VA = 999936
NT = V - VA
PAD = 1000448
CHW = 32256
NLCH = VA // CHW


def _sc_linearize(u_tabT, r_tabT):
    mesh = plsc.VectorSubcoreMesh(core_axis_name="c", subcore_axis_name="s")

    @functools.partial(
        pl.kernel,
        mesh=mesh,
        out_type=(
            jax.ShapeDtypeStruct((E * PAD,), jnp.float32),
            jax.ShapeDtypeStruct((E * PAD,), jnp.float32),
        ),
        scratch_types=[
            pltpu.VMEM((2, CHW), jnp.float32),
            pltpu.SemaphoreType.DMA,
            pltpu.SemaphoreType.DMA,
            pltpu.SemaphoreType.DMA,
        ],
        compiler_params=pltpu.CompilerParams(needs_layout_passes=False),
    )
    def k(u_tab_hbm, r_tab_hbm, u_out_hbm, r_out_hbm, buf, sem_in, s_w0, s_w1):
        wid = lax.axis_index("s") * NC + lax.axis_index("c")
        lane = lax.rem(wid, E)
        wsems = (s_w0, s_w1)

        def do_table(tab, out):
            reads = [None, None]
            writes = [None, None]
            reads[0] = pltpu.async_copy(
                tab.at[lane, pl.ds(0, CHW)], buf.at[0], sem_in)
            for c in range(NLCH):
                b = c % 2
                reads[b].wait()
                if c + 1 < NLCH:
                    nb = (c + 1) % 2
                    if writes[nb] is not None:
                        writes[nb].wait()
                    reads[nb] = pltpu.async_copy(
                        tab.at[lane, pl.ds((c + 1) * CHW, CHW)], buf.at[nb], sem_in)
                writes[b] = pltpu.async_copy(
                    buf.at[b], out.at[pl.ds(lane * PAD + c * CHW, CHW)], wsems[b])
            for w in writes:
                if w is not None:
                    w.wait()

        @pl.when(wid < E)
        def _():
            do_table(u_tab_hbm, u_out_hbm)

        @pl.when(wid >= E)
        def _():
            do_table(r_tab_hbm, r_out_hbm)

    return k(u_tabT, r_tabT)


def _sc_gather_dot(u_idx2d, r_idx2d, u_lin, r_lin, u_tail, r_tail,
                   u_bias, r_bias):
    mesh = plsc.VectorSubcoreMesh(core_axis_name="c", subcore_axis_name="s")

    @functools.partial(
        pl.kernel,
        mesh=mesh,
        out_type=(
            jax.ShapeDtypeStruct((NW * E,), jnp.float32),
            jax.ShapeDtypeStruct((B,), jnp.float32),
        ),
        scratch_types=[
            pltpu.VMEM((E, BPW), jnp.int32),
            pltpu.VMEM((E, BPW), jnp.int32),
            pltpu.VMEM((NCH, CH), jnp.int32),
            pltpu.VMEM((NCH, CH), jnp.int32),
            pltpu.VMEM((E, BPW), jnp.float32),
            pltpu.VMEM((E, BPW), jnp.float32),
            pltpu.VMEM((NT, E), jnp.float32),
            pltpu.VMEM((NT, E), jnp.float32),
            pltpu.VMEM((BPW,), jnp.float32),
            pltpu.VMEM((BPW,), jnp.float32),
            pltpu.VMEM((BPW,), jnp.float32),
            pltpu.VMEM((E,), jnp.float32),
            pltpu.SemaphoreType.DMA,
            pltpu.SemaphoreType.DMA,
            pltpu.SemaphoreType.DMA,
            pltpu.SemaphoreType.DMA,
            pltpu.SemaphoreType.DMA,
        ],
        compiler_params=pltpu.CompilerParams(
            use_tc_tiling_on_sc=False, needs_layout_passes=False),
    )
    def k(u_idx_hbm, r_idx_hbm, u_lin_hbm, r_lin_hbm,
          u_tail_hbm, r_tail_hbm, u_bias_hbm, r_bias_hbm, partial_hbm, ubrb_hbm,
          lidx_u, lidx_r, idx_u, idx_r, u_vals, r_vals, ut_v, rt_v, ub_v, rb_v,
          ubrb_v, acc_v, sem, sem_c0, sem_c1, sem_c2, sem_c3):
        wid = lax.axis_index("s") * NC + lax.axis_index("c")
        base = pl.multiple_of(wid * BPW, 8)
        row0 = wid * NCH

        pltpu.sync_copy(u_idx_hbm.at[pl.ds(row0, NCH)], idx_u)
        pltpu.sync_copy(r_idx_hbm.at[pl.ds(row0, NCH)], idx_r)
        pltpu.sync_copy(u_tail_hbm, ut_v)
        pltpu.sync_copy(r_tail_hbm, rt_v)

        for j in range(NCH):
            for i8 in range(CH // E):
                gi = pl.ds(i8 * E, E)
                col = pl.ds(j * CH + i8 * E, E)
                iu = jnp.minimum(idx_u[j, gi], VA - 1)
                ir = jnp.minimum(idx_r[j, gi], VA - 1)
                for l in range(E):
                    lidx_u[l, col] = iu + (l * PAD)
                    lidx_r[l, col] = ir + (l * PAD)

        csems = (sem_c0, sem_c1, sem_c2, sem_c3)
        bias_copies = []
        waves = []
        for j in range(NCH):
            sl = pl.ds(j * CH, CH)
            bias_copies.append(
                pltpu.async_copy(u_bias_hbm.at[idx_u.at[j]], ub_v.at[sl], sem))
            bias_copies.append(
                pltpu.async_copy(r_bias_hbm.at[idx_r.at[j]], rb_v.at[sl], sem))
            wave = []
            for l in range(E):
                wave.append(pltpu.async_copy(
                    u_lin_hbm.at[lidx_u.at[l, sl]], u_vals.at[l, sl], csems[j]))
                wave.append(pltpu.async_copy(
                    r_lin_hbm.at[lidx_r.at[l, sl]], r_vals.at[l, sl], csems[j]))
            waves.append(wave)

        acc = jnp.zeros((E,), jnp.float32)
        for j in range(NCH):
            for c in waves[j]:
                c.wait()
            for i8 in range(CH // E):
                g = pl.ds(j * CH + i8 * E, E)
                gi = pl.ds(i8 * E, E)
                iu = idx_u[j, gi]
                ir = idx_r[j, gi]
                mu = iu >= VA
                mr = ir >= VA
                tu = jnp.maximum(iu - VA, 0)
                tr = jnp.maximum(ir - VA, 0)
                for l in range(E):
                    lcol = jnp.full((E,), l, jnp.int32)
                    uv = jnp.where(mu, plsc.load_gather(ut_v, [tu, lcol]),
                                   u_vals[l, g])
                    rv = jnp.where(mr, plsc.load_gather(rt_v, [tr, lcol]),
                                   r_vals[l, g])
                    acc = acc + uv * rv
        acc_v[...] = acc
        pltpu.sync_copy(acc_v, partial_hbm.at[pl.ds(pl.multiple_of(wid * E, 8), E)])

        for c in bias_copies:
            c.wait()
        for i in range(BPW // E):
            sl = pl.ds(i * E, E)
            ubrb_v[sl] = ub_v[sl] + rb_v[sl]
        pltpu.sync_copy(ubrb_v, ubrb_hbm.at[pl.ds(base, BPW)])

    return k(u_idx2d, r_idx2d, u_lin, r_lin, u_tail, r_tail, u_bias, r_bias)


def _tc_finish(partials_2d, ubrb_2d):
    def body(p_ref, x_ref, o_ref):
        s = jnp.sum(p_ref[...])
        o_ref[...] = jax.nn.sigmoid(x_ref[...] + s)

    return pl.pallas_call(
        body,
        out_shape=jax.ShapeDtypeStruct(ubrb_2d.shape, jnp.float32),
    )(partials_2d, ubrb_2d)


def kernel(inputs, user_embedding, user_bias, resto_embedding, resto_bias):
    idx = inputs.astype(jnp.int32)
    u_idx = idx[:, 0]
    r_idx = idx[:, 1]
    u_lin, r_lin = _sc_linearize(user_embedding.T, resto_embedding.T)
    partials, ubrb = _sc_gather_dot(
        u_idx.reshape(B // CH, CH), r_idx.reshape(B // CH, CH),
        u_lin, r_lin,
        user_embedding[VA:, :], resto_embedding[VA:, :],
        user_bias.reshape(-1), resto_bias.reshape(-1))
    out = _tc_finish(partials.reshape(NW * E // 128, 128), ubrb.reshape(B // 128, 128))
    return out.reshape(B, 1)
